# Initial kernel scaffold; baseline (speedup 1.0000x reference)
import functools

import jax
import jax.numpy as jnp
from jax import lax
from jax.experimental import pallas as pl
from jax.experimental.pallas import tpu as pltpu
from jax.experimental.pallas import tpu_sc as plsc

_C = 80


def _degree_kernel(E, NP):
    RC = NP // 128
    EPW = E // 16
    NCH = EPW // _C
    mesh = plsc.VectorSubcoreMesh(core_axis_name="c", subcore_axis_name="s")

    @functools.partial(
        pl.kernel,
        out_type=jax.ShapeDtypeStruct((2, NP), jnp.float32),
        mesh=mesh,
        scratch_types=[
            pltpu.VMEM((_C,), jnp.int32),
            pltpu.VMEM((_C,), jnp.float32),
            pltpu.VMEM((128,), jnp.float32),
            pltpu.VMEM_SHARED((NP,), jnp.float32),
        ],
    )
    def deg(ei, deg_out, idx_v, ones_v, zeros_v, deg_sh):
        c = lax.axis_index("c")
        s = lax.axis_index("s")
        one16 = jnp.ones((16,), jnp.float32)
        zero16 = jnp.zeros((16,), jnp.float32)
        for i in range(_C // 16):
            ones_v[pl.ds(i * 16, 16)] = one16
        for i in range(8):
            zeros_v[pl.ds(i * 16, 16)] = zero16

        @pl.when(s == 0)
        def _():
            def zbody(k, carry):
                pltpu.sync_copy(zeros_v, deg_sh.at[pl.ds(k * 128, 128)])
                return carry
            lax.fori_loop(0, RC, zbody, 0)

        plsc.subcore_barrier()

        def hbody(j, carry):
            off = s * EPW + j * _C
            pltpu.sync_copy(ei.at[c, pl.ds(off, _C)], idx_v)
            pltpu.sync_copy(ones_v, deg_sh.at[idx_v], add=True)
            return carry
        lax.fori_loop(0, NCH, hbody, 0)

        plsc.subcore_barrier()

        def obody(j, carry):
            ck = s + j * 16
            pltpu.sync_copy(deg_sh.at[pl.ds(ck * 128, 128)],
                            deg_out.at[c, pl.ds(ck * 128, 128)])
            return carry
        lax.fori_loop(0, (RC - s + 15) // 16, obody, 0)

    return deg


def _aggregate_kernel(N, D, E):
    EPW = E // 32
    NCH = EPW // _C
    RCH = N // _C
    mesh = plsc.VectorSubcoreMesh(core_axis_name="c", subcore_axis_name="s")

    @functools.partial(
        pl.kernel,
        out_type=jax.ShapeDtypeStruct((2, N, D), jnp.float32),
        mesh=mesh,
        scratch_types=[
            pltpu.VMEM((_C,), jnp.int32),
            pltpu.VMEM((_C,), jnp.int32),
            pltpu.VMEM((_C, D), jnp.float32),
            pltpu.VMEM_SHARED((N, D), jnp.float32),
            pltpu.SemaphoreType.DMA,
        ],
    )
    def agg(h, ei, out, src_v, dst_v, rows_v, acc_sh, sem):
        c = lax.axis_index("c")
        s = lax.axis_index("s")
        w = c * 16 + s
        zero16 = jnp.zeros((16,), jnp.float32)
        for r in range(_C):
            for i in range(D // 16):
                rows_v[r, pl.ds(i * 16, 16)] = zero16

        def zbody(j, carry):
            ck = s + j * 16
            pltpu.sync_copy(rows_v, acc_sh.at[pl.ds(ck * _C, _C)])
            return carry
        lax.fori_loop(0, (RCH - s + 15) // 16, zbody, 0)

        plsc.subcore_barrier()

        def ebody(j, carry):
            off = w * EPW + j * _C
            pltpu.sync_copy(ei.at[0, pl.ds(off, _C)], src_v)
            pltpu.sync_copy(ei.at[1, pl.ds(off, _C)], dst_v)
            pltpu.async_copy(h.at[src_v], rows_v, sem).wait()
            pltpu.sync_copy(rows_v, acc_sh.at[dst_v], add=True)
            return carry
        lax.fori_loop(0, NCH, ebody, 0)

        plsc.subcore_barrier()

        def obody(j, carry):
            ck = s + j * 16
            pltpu.sync_copy(acc_sh.at[pl.ds(ck * _C, _C)],
                            out.at[c, pl.ds(ck * _C, _C)])
            return carry
        lax.fori_loop(0, (RCH - s + 15) // 16, obody, 0)

    return agg


def _scale_body(f_ref, d_ref, h_ref):
    h_ref[...] = f_ref[...] * lax.rsqrt(jnp.maximum(d_ref[...], 1.0))


def _finish_body(p_ref, w_ref, d_ref, b_ref, o_ref):
    acc = p_ref[0] + p_ref[1]
    r = lax.dot_general(acc, w_ref[...], (((1,), (0,)), ((), ())),
                        precision=lax.Precision.HIGHEST,
                        preferred_element_type=jnp.float32)
    o_ref[...] = r * lax.rsqrt(jnp.maximum(d_ref[...], 1.0)) + b_ref[...]


def kernel(feat, edge_index, W, b):
    N, D = feat.shape
    E = edge_index.shape[1]
    NP = ((N + 127) // 128) * 128

    deg = _degree_kernel(E, NP)(edge_index)
    out_deg = deg[0, :N].reshape(N, 1)
    in_deg = deg[1, :N].reshape(N, 1)

    h = pl.pallas_call(
        _scale_body,
        out_shape=jax.ShapeDtypeStruct((N, D), jnp.float32),
    )(feat, out_deg)

    parts = _aggregate_kernel(N, D, E)(h, edge_index)

    rst = pl.pallas_call(
        _finish_body,
        out_shape=jax.ShapeDtypeStruct((N, W.shape[1]), jnp.float32),
    )(parts, W, in_deg, b.reshape(1, W.shape[1]))
    return rst


# speedup vs baseline: 5.0401x; 5.0401x over previous
import functools

import jax
import jax.numpy as jnp
from jax import lax
from jax.experimental import pallas as pl
from jax.experimental.pallas import tpu as pltpu
from jax.experimental.pallas import tpu_sc as plsc

_C = 80


def _degree_kernel(E, NP):
    RC = NP // 128
    EPW = E // 16
    NCH = EPW // _C
    mesh = plsc.VectorSubcoreMesh(core_axis_name="c", subcore_axis_name="s")

    @functools.partial(
        pl.kernel,
        out_type=jax.ShapeDtypeStruct((2 * NP,), jnp.float32),
        mesh=mesh,
        scratch_types=[
            pltpu.VMEM((_C,), jnp.int32),
            pltpu.VMEM((_C,), jnp.float32),
            pltpu.VMEM((128,), jnp.float32),
            pltpu.VMEM_SHARED((NP,), jnp.float32),
        ],
    )
    def deg(ei, deg_out, idx_v, ones_v, zeros_v, deg_sh):
        c = lax.axis_index("c")
        s = lax.axis_index("s")
        one16 = jnp.ones((16,), jnp.float32)
        zero16 = jnp.zeros((16,), jnp.float32)
        for i in range(_C // 16):
            ones_v[pl.ds(i * 16, 16)] = one16
        for i in range(8):
            zeros_v[pl.ds(i * 16, 16)] = zero16

        @pl.when(s == 0)
        def _():
            def zbody(k, carry):
                pltpu.sync_copy(zeros_v, deg_sh.at[pl.ds(k * 128, 128)])
                return carry
            lax.fori_loop(0, RC, zbody, 0)

        plsc.subcore_barrier()

        def hbody(j, carry):
            off = c * E + s * EPW + j * _C
            pltpu.sync_copy(ei.at[pl.ds(off, _C)], idx_v)
            pltpu.sync_copy(ones_v, deg_sh.at[idx_v], add=True)
            return carry
        lax.fori_loop(0, NCH, hbody, 0)

        plsc.subcore_barrier()

        def obody(j, carry):
            ck = s + j * 16
            pltpu.sync_copy(deg_sh.at[pl.ds(ck * 128, 128)],
                            deg_out.at[pl.ds(c * NP + ck * 128, 128)])
            return carry
        lax.fori_loop(0, (RC - s + 15) // 16, obody, 0)

    return deg


def _aggregate_kernel(N, D, E):
    EPW = E // 32
    NCH = EPW // _C
    RCH = N // _C
    mesh = plsc.VectorSubcoreMesh(core_axis_name="c", subcore_axis_name="s")

    @functools.partial(
        pl.kernel,
        out_type=jax.ShapeDtypeStruct((2, N, D), jnp.float32),
        mesh=mesh,
        scratch_types=[
            pltpu.VMEM((_C,), jnp.int32),
            pltpu.VMEM((_C,), jnp.int32),
            pltpu.VMEM((_C, D), jnp.float32),
            pltpu.VMEM_SHARED((N, D), jnp.float32),
            pltpu.SemaphoreType.DMA,
        ],
    )
    def agg(h, ei, out, src_v, dst_v, rows_v, acc_sh, sem):
        c = lax.axis_index("c")
        s = lax.axis_index("s")
        w = c * 16 + s
        zero16 = jnp.zeros((16,), jnp.float32)
        for r in range(_C):
            for i in range(D // 16):
                rows_v[r, pl.ds(i * 16, 16)] = zero16

        def zbody(j, carry):
            ck = s + j * 16
            pltpu.sync_copy(rows_v, acc_sh.at[pl.ds(ck * _C, _C)])
            return carry
        lax.fori_loop(0, (RCH - s + 15) // 16, zbody, 0)

        plsc.subcore_barrier()

        def ebody(j, carry):
            off = w * EPW + j * _C
            pltpu.sync_copy(ei.at[pl.ds(off, _C)], src_v)
            pltpu.sync_copy(ei.at[pl.ds(E + off, _C)], dst_v)
            pltpu.async_copy(h.at[src_v], rows_v, sem).wait()
            pltpu.sync_copy(rows_v, acc_sh.at[dst_v], add=True)
            return carry
        lax.fori_loop(0, NCH, ebody, 0)

        plsc.subcore_barrier()

        def obody(j, carry):
            ck = s + j * 16
            pltpu.sync_copy(acc_sh.at[pl.ds(ck * _C, _C)],
                            out.at[c, pl.ds(ck * _C, _C)])
            return carry
        lax.fori_loop(0, (RCH - s + 15) // 16, obody, 0)

    return agg


def _scale_body(f_ref, d_ref, h_ref):
    h_ref[...] = f_ref[...] * lax.rsqrt(jnp.maximum(d_ref[...], 1.0))


def _finish_body(p_ref, w_ref, d_ref, b_ref, o_ref):
    acc = p_ref[0] + p_ref[1]
    r = lax.dot_general(acc, w_ref[...], (((1,), (0,)), ((), ())),
                        precision=lax.Precision.HIGHEST,
                        preferred_element_type=jnp.float32)
    o_ref[...] = r * lax.rsqrt(jnp.maximum(d_ref[...], 1.0)) + b_ref[...]


def kernel(feat, edge_index, W, b):
    N, D = feat.shape
    E = edge_index.shape[1]
    NP = ((N + 127) // 128) * 128

    ei_flat = edge_index.reshape(2 * E)
    deg = _degree_kernel(E, NP)(ei_flat)
    out_deg = deg[:N].reshape(N, 1)
    in_deg = deg[NP:NP + N].reshape(N, 1)

    h = pl.pallas_call(
        _scale_body,
        out_shape=jax.ShapeDtypeStruct((N, D), jnp.float32),
    )(feat, out_deg)

    parts = _aggregate_kernel(N, D, E)(h, ei_flat)

    rst = pl.pallas_call(
        _finish_body,
        out_shape=jax.ShapeDtypeStruct((N, W.shape[1]), jnp.float32),
    )(parts, W, in_deg, b.reshape(1, W.shape[1]))
    return rst
